# baseline (speedup 1.0000x reference)
import functools

import jax
import jax.numpy as jnp
from jax import lax
from jax.experimental import pallas as pl
from jax.experimental.pallas import tpu as pltpu
from jax.experimental.pallas import tpu_sc as plsc

NUM_CORES = 2
NUM_SUBCORES = 16
NUM_WORKERS = NUM_CORES * NUM_SUBCORES

LANES = 16
T_CHUNK = 5


def _sc_embed_t(idx, table):
    batch, hist = idx.shape
    vocab, dim = table.shape
    b_per_worker = batch // NUM_WORKERS
    n_bvecs = b_per_worker // LANES
    n_chunks = hist // T_CHUNK
    assert n_chunks % 2 == 0
    chunk_rows = T_CHUNK * dim

    mesh = plsc.VectorSubcoreMesh(core_axis_name="c", subcore_axis_name="s")

    @functools.partial(
        pl.kernel,
        out_type=jax.ShapeDtypeStruct((hist * dim, batch), jnp.float32),
        mesh=mesh,
        scratch_types=[
            pltpu.VMEM((vocab, dim), jnp.float32),
            pltpu.VMEM((b_per_worker, hist), jnp.int32),
            pltpu.VMEM((chunk_rows, b_per_worker), jnp.float32),
            pltpu.VMEM((chunk_rows, b_per_worker), jnp.float32),
            pltpu.SemaphoreType.DMA,
            pltpu.SemaphoreType.DMA,
        ],
        compiler_params=pltpu.CompilerParams(use_tc_tiling_on_sc=False,
                                             needs_layout_passes=False),
    )
    def k(table_hbm, idx_hbm, out_hbm, table_v, idx_v, buf0, buf1,
          o0sem, o1sem):
        wid = lax.axis_index("s") * NUM_CORES + lax.axis_index("c")
        b_base = wid * b_per_worker

        pltpu.sync_copy(table_hbm, table_v)
        pltpu.sync_copy(idx_hbm.at[pl.ds(b_base, b_per_worker)], idx_v)

        lane = jax.lax.iota(jnp.int32, LANES)

        def compute(g, buf):
            def t_body(tt, carry):
                t = g * T_CHUNK + tt
                tvec = jnp.full((LANES,), t, jnp.int32)
                for v in range(n_bvecs):
                    bvec = lane + (v * LANES)
                    idx16 = plsc.load_gather(idx_v, [bvec, tvec])
                    for d in range(dim):
                        dvec = jnp.full((LANES,), d, jnp.int32)
                        vals = plsc.load_gather(table_v, [idx16, dvec])
                        buf[tt * dim + d, pl.ds(v * LANES, LANES)] = vals
                return carry

            lax.fori_loop(0, T_CHUNK, t_body, 0)

        def fire_out(g, buf, sem):
            pltpu.async_copy(
                buf,
                out_hbm.at[pl.ds(g * chunk_rows, chunk_rows),
                           pl.ds(b_base, b_per_worker)],
                sem,
            )

        def wait_out(buf, sem):
            pltpu.make_async_copy(
                buf,
                out_hbm.at[pl.ds(0, chunk_rows), pl.ds(0, b_per_worker)],
                sem,
            ).wait()

        def body(i, carry):
            @pl.when(i > 0)
            def _():
                wait_out(buf0, o0sem)
                wait_out(buf1, o1sem)

            compute(2 * i, buf0)
            fire_out(2 * i, buf0, o0sem)
            compute(2 * i + 1, buf1)
            fire_out(2 * i + 1, buf1, o1sem)
            return carry

        lax.fori_loop(0, n_chunks // 2, body, 0)
        wait_out(buf0, o0sem)
        wait_out(buf1, o1sem)

    return k(table, idx)


def kernel(inputs, table):
    b, t = inputs.shape
    vocab, dim = table.shape
    out_t = _sc_embed_t(inputs.astype(jnp.int32), table)
    return out_t.reshape(t, dim, b).transpose(2, 0, 1)


# speedup vs baseline: 1.6515x; 1.6515x over previous
import functools

import jax
import jax.numpy as jnp
from jax import lax
from jax.experimental import pallas as pl
from jax.experimental.pallas import tpu as pltpu
from jax.experimental.pallas import tpu_sc as plsc

NUM_CORES = 2
NUM_SUBCORES = 16
NUM_WORKERS = NUM_CORES * NUM_SUBCORES

LANES = 16
ROW_CHUNK = 8
PAIR_ILV = 2


def _sc_embed_t(idx_t, table):
    hist, batch = idx_t.shape
    vocab, dim = table.shape
    n_rows = hist * dim
    rows_per_worker = n_rows // NUM_WORKERS
    n_chunks = rows_per_worker // ROW_CHUNK
    assert n_chunks % 2 == 0
    n_bvecs = batch // LANES
    t_span = rows_per_worker // dim + 1

    mesh = plsc.VectorSubcoreMesh(core_axis_name="c", subcore_axis_name="s")

    @functools.partial(
        pl.kernel,
        out_type=jax.ShapeDtypeStruct((n_rows, batch), jnp.float32),
        mesh=mesh,
        scratch_types=[
            pltpu.VMEM((vocab, dim), jnp.float32),
            pltpu.VMEM((t_span, batch), jnp.int32),
            pltpu.VMEM((ROW_CHUNK, batch), jnp.float32),
            pltpu.VMEM((ROW_CHUNK, batch), jnp.float32),
            pltpu.SemaphoreType.DMA,
            pltpu.SemaphoreType.DMA,
        ],
        compiler_params=pltpu.CompilerParams(use_tc_tiling_on_sc=False,
                                             needs_layout_passes=False),
    )
    def k(table_hbm, idxt_hbm, out_hbm, table_v, idx_v, buf0, buf1,
          o0sem, o1sem):
        wid = lax.axis_index("s") * NUM_CORES + lax.axis_index("c")
        row_base = wid * rows_per_worker
        t_lo = jnp.minimum(row_base // dim, hist - t_span)

        pltpu.sync_copy(table_hbm, table_v)
        pltpu.sync_copy(idxt_hbm.at[pl.ds(t_lo, t_span)], idx_v)

        def compute(g, buf):
            r0 = row_base + g * ROW_CHUNK
            t_local = r0 // dim - t_lo
            d0 = lax.rem(r0, dim)

            @plsc.parallel_loop(0, n_bvecs, step=PAIR_ILV)
            def b_body(v0):
                idxs = []
                for p in range(PAIR_ILV):
                    b0 = (v0 + p) * LANES
                    idxs.append(idx_v[t_local, pl.ds(b0, LANES)])
                for d in range(ROW_CHUNK):
                    dvec = jnp.full((LANES,), d0 + d, jnp.int32)
                    for p in range(PAIR_ILV):
                        vals = plsc.load_gather(table_v, [idxs[p], dvec])
                        buf[d, pl.ds((v0 + p) * LANES, LANES)] = vals

        def fire_out(g, buf, sem):
            pltpu.async_copy(
                buf,
                out_hbm.at[pl.ds(row_base + g * ROW_CHUNK, ROW_CHUNK)],
                sem,
            )

        def wait_out(buf, sem):
            pltpu.make_async_copy(
                buf, out_hbm.at[pl.ds(0, ROW_CHUNK)], sem).wait()

        def body(i, carry):
            @pl.when(i > 0)
            def _():
                wait_out(buf0, o0sem)
                wait_out(buf1, o1sem)

            compute(2 * i, buf0)
            fire_out(2 * i, buf0, o0sem)
            compute(2 * i + 1, buf1)
            fire_out(2 * i + 1, buf1, o1sem)
            return carry

        lax.fori_loop(0, n_chunks // 2, body, 0)
        wait_out(buf0, o0sem)
        wait_out(buf1, o1sem)

    return k(table, idx_t)


def kernel(inputs, table):
    b, t = inputs.shape
    vocab, dim = table.shape
    idx_t = inputs.astype(jnp.int32).T
    out_t = _sc_embed_t(idx_t, table)
    return out_t.reshape(t, dim, b).transpose(2, 0, 1)


# speedup vs baseline: 2.9542x; 1.7887x over previous
import functools

import jax
import jax.numpy as jnp
from jax import lax
from jax.experimental import pallas as pl
from jax.experimental.pallas import tpu as pltpu
from jax.experimental.pallas import tpu_sc as plsc

NUM_CORES = 2
NUM_SUBCORES = 16
NUM_WORKERS = NUM_CORES * NUM_SUBCORES

LANES = 16
SUBLANES = 8
PAIR_ILV = 2


def _sc_embed_tiled(idx_t, table_p):
    hist, batch = idx_t.shape
    vocab, dim_p = table_p.shape
    dim = dim_p - 1
    bcol = batch // NUM_WORKERS
    n_bvecs = bcol // LANES
    d_hi = dim // SUBLANES
    assert hist % 2 == 0

    mesh = plsc.VectorSubcoreMesh(core_axis_name="c", subcore_axis_name="s")

    @functools.partial(
        pl.kernel,
        out_type=jax.ShapeDtypeStruct(
            (hist * d_hi, NUM_WORKERS, SUBLANES, 128), jnp.float32),
        mesh=mesh,
        scratch_types=[
            pltpu.VMEM((vocab, dim_p), jnp.float32),
            pltpu.VMEM((hist, bcol), jnp.int32),
            pltpu.VMEM((d_hi, SUBLANES, 128), jnp.float32),
            pltpu.VMEM((d_hi, SUBLANES, 128), jnp.float32),
            pltpu.SemaphoreType.DMA,
            pltpu.SemaphoreType.DMA,
        ],
        compiler_params=pltpu.CompilerParams(use_tc_tiling_on_sc=False,
                                             needs_layout_passes=False),
    )
    def k(table_hbm, idxt_hbm, out_hbm, table_v, idx_v, buf0, buf1,
          o0sem, o1sem):
        wid = lax.axis_index("s") * NUM_CORES + lax.axis_index("c")

        pltpu.sync_copy(table_hbm, table_v)
        pltpu.sync_copy(idxt_hbm.at[:, pl.ds(wid * bcol, bcol)], idx_v)

        def compute(t, buf):
            @plsc.parallel_loop(0, n_bvecs, step=PAIR_ILV)
            def v_body(v0):
                idxs = []
                for p in range(PAIR_ILV):
                    idxs.append(idx_v[t, pl.ds((v0 + p) * LANES, LANES)])
                for d in range(dim):
                    dvec = jnp.full((LANES,), d, jnp.int32)
                    for p in range(PAIR_ILV):
                        vals = plsc.load_gather(table_v, [idxs[p], dvec])
                        buf[d // SUBLANES, d % SUBLANES,
                            pl.ds((v0 + p) * LANES, LANES)] = vals

        def fire_out(t, buf, sem):
            pltpu.async_copy(
                buf, out_hbm.at[pl.ds(t * d_hi, d_hi), wid], sem)

        def wait_out(buf, sem):
            pltpu.make_async_copy(
                buf, out_hbm.at[pl.ds(0, d_hi), 0], sem).wait()

        def body(i, carry):
            @pl.when(i > 0)
            def _():
                wait_out(buf0, o0sem)
                wait_out(buf1, o1sem)

            compute(2 * i, buf0)
            fire_out(2 * i, buf0, o0sem)
            compute(2 * i + 1, buf1)
            fire_out(2 * i + 1, buf1, o1sem)
            return carry

        lax.fori_loop(0, hist // 2, body, 0)
        wait_out(buf0, o0sem)
        wait_out(buf1, o1sem)

    return k(table_p, idx_t)


def kernel(inputs, table):
    b, t = inputs.shape
    vocab, dim = table.shape
    idx_t = inputs.astype(jnp.int32).T
    table_p = jnp.pad(table, ((0, 0), (0, 1)))
    out4 = _sc_embed_tiled(idx_t, table_p)
    out5 = out4.reshape(t, dim // 8, b // 128, 8, 128)
    return out5.transpose(2, 4, 0, 1, 3).reshape(b, t, dim)


# speedup vs baseline: 8.8792x; 3.0057x over previous
import functools

import jax
import jax.numpy as jnp
from jax import lax
from jax.experimental import pallas as pl
from jax.experimental.pallas import tpu as pltpu
from jax.experimental.pallas import tpu_sc as plsc

NUM_CORES = 2
NUM_SUBCORES = 16
NUM_WORKERS = NUM_CORES * NUM_SUBCORES

LANES = 16
SUBLANES = 8
PAIR_ILV = 2
D_ILV = 4


def _sc_embed_tiled(idx_t, table_flat, dim):
    hist, batch = idx_t.shape
    stride = dim + 1
    bcol = batch // NUM_WORKERS
    n_bvecs = bcol // LANES
    d_hi = dim // SUBLANES
    assert hist % 2 == 0 and dim % (SUBLANES * D_ILV) == 0

    mesh = plsc.VectorSubcoreMesh(core_axis_name="c", subcore_axis_name="s")

    @functools.partial(
        pl.kernel,
        out_type=jax.ShapeDtypeStruct(
            (hist * d_hi, NUM_WORKERS, SUBLANES, 128), jnp.float32),
        mesh=mesh,
        scratch_types=[
            pltpu.VMEM(table_flat.shape, jnp.float32),
            pltpu.VMEM((hist, bcol), jnp.int32),
            pltpu.VMEM((d_hi, SUBLANES, 128), jnp.float32),
            pltpu.VMEM((d_hi, SUBLANES, 128), jnp.float32),
            pltpu.SemaphoreType.DMA,
            pltpu.SemaphoreType.DMA,
        ],
        compiler_params=pltpu.CompilerParams(use_tc_tiling_on_sc=False,
                                             needs_layout_passes=False),
    )
    def k(table_hbm, idxt_hbm, out_hbm, table_v, idx_v, buf0, buf1,
          o0sem, o1sem):
        wid = lax.axis_index("s") * NUM_CORES + lax.axis_index("c")

        pltpu.sync_copy(table_hbm, table_v)
        pltpu.sync_copy(idxt_hbm.at[:, pl.ds(wid * bcol, bcol)], idx_v)

        def compute(t, buf):
            @plsc.parallel_loop(0, n_bvecs, step=PAIR_ILV)
            def v_body(v0):
                bases = []
                for p in range(PAIR_ILV):
                    idx16 = idx_v[t, pl.ds((v0 + p) * LANES, LANES)]
                    bases.append(idx16 * stride)
                for d0 in range(0, dim, D_ILV):
                    vals = [
                        plsc.load_gather(table_v, [bases[p] + (d0 + j)])
                        for j in range(D_ILV) for p in range(PAIR_ILV)
                    ]
                    i = 0
                    for j in range(D_ILV):
                        d = d0 + j
                        for p in range(PAIR_ILV):
                            buf[d // SUBLANES, d % SUBLANES,
                                pl.ds((v0 + p) * LANES, LANES)] = vals[i]
                            i += 1

        def fire_out(t, buf, sem):
            pltpu.async_copy(
                buf, out_hbm.at[pl.ds(t * d_hi, d_hi), wid], sem)

        def wait_out(buf, sem):
            pltpu.make_async_copy(
                buf, out_hbm.at[pl.ds(0, d_hi), 0], sem).wait()

        def body(i, carry):
            @pl.when(i > 0)
            def _():
                wait_out(buf0, o0sem)
                wait_out(buf1, o1sem)

            compute(2 * i, buf0)
            fire_out(2 * i, buf0, o0sem)
            compute(2 * i + 1, buf1)
            fire_out(2 * i + 1, buf1, o1sem)
            return carry

        lax.fori_loop(0, hist // 2, body, 0)
        wait_out(buf0, o0sem)
        wait_out(buf1, o1sem)

    return k(table_flat, idx_t)


def kernel(inputs, table):
    b, t = inputs.shape
    vocab, dim = table.shape
    idx_t = inputs.astype(jnp.int32).T
    table_flat = jnp.pad(table, ((0, 0), (0, 1))).reshape(-1)
    out4 = _sc_embed_tiled(idx_t, table_flat, dim)
    out5 = out4.reshape(t, dim // 8, b // 128, 8, 128)
    return out5.transpose(2, 4, 0, 1, 3).reshape(b, t, dim)
